# baseline (speedup 1.0000x reference)
import functools

import jax
import jax.numpy as jnp
from jax import lax
from jax.experimental import pallas as pl
from jax.experimental.pallas import tpu as pltpu
from jax.experimental.pallas import tpu_sc as plsc

_E = 64
_G = 8
_GS = 8
_TOPK = 8
_TOPKG = 4
_SCALE = 2.5
_L = 16


def _gate_tc_body(w_ref, x_ref, b_ref, s_ref, hv_ref, hi_ref):
    logits = lax.dot_general(
        w_ref[...], x_ref[...],
        dimension_numbers=(((1,), (1,)), ((), ())),
        preferred_element_type=jnp.float32,
    )
    s = jax.nn.sigmoid(logits) + b_ref[...]
    s_ref[...] = s
    bt = s.shape[1]
    g3 = s.reshape(_G, _GS, bt)
    iota_j = lax.broadcasted_iota(jnp.int32, (_G, _GS, bt), 1)
    m1 = jnp.max(g3, axis=1)
    eq = g3 == m1[:, None, :]
    fi = jnp.min(jnp.where(eq, iota_j, _GS), axis=1)
    first = eq & (iota_j == fi[:, None, :])
    m2 = jnp.max(jnp.where(first, -1.0, g3), axis=1)
    gsc = m1 + m2
    iota_g = lax.broadcasted_iota(jnp.int32, (_G, bt), 0)
    sel = None
    for _ in range(_TOPKG):
        gm = jnp.max(gsc, axis=0)
        eqg = gsc == gm[None, :]
        fg = jnp.min(jnp.where(eqg, iota_g, _G), axis=0)
        hit = iota_g == fg[None, :]
        sel = hit if sel is None else (sel | hit)
        gsc = jnp.where(hit, -1.0, gsc)
    hv_ref[...] = jnp.where(sel, m1, -1.0)
    hi_ref[...] = iota_g * _GS + fi


def _gate_scores_t(x, w, b, block_t=1024):
    t, h = x.shape
    return pl.pallas_call(
        _gate_tc_body,
        grid=(t // block_t,),
        in_specs=[
            pl.BlockSpec((_E, h), lambda i: (0, 0)),
            pl.BlockSpec((block_t, h), lambda i: (i, 0)),
            pl.BlockSpec((_E, 1), lambda i: (0, 0)),
        ],
        out_specs=[
            pl.BlockSpec((_E, block_t), lambda i: (0, i)),
            pl.BlockSpec((_G, block_t), lambda i: (0, i)),
            pl.BlockSpec((_G, block_t), lambda i: (0, i)),
        ],
        out_shape=[
            jax.ShapeDtypeStruct((_E, t), jnp.float32),
            jax.ShapeDtypeStruct((_G, t), jnp.float32),
            jax.ShapeDtypeStruct((_G, t), jnp.int32),
        ],
    )(w, x, b.reshape(_E, 1))


def _route_sc(scores_t, hv0, hi0, bias):
    t = scores_t.shape[1]
    info = plsc.get_sparse_core_info()
    nc, ns = info.num_cores, info.num_subcores
    nw = nc * ns
    tw = t // nw
    nslab = tw // _L
    mesh = plsc.VectorSubcoreMesh(core_axis_name="c", subcore_axis_name="s")

    @functools.partial(
        pl.kernel,
        mesh=mesh,
        compiler_params=pltpu.CompilerParams(needs_layout_passes=False),
        out_type=[
            jax.ShapeDtypeStruct((_TOPK, t), jnp.int32),
            jax.ShapeDtypeStruct((_TOPK, t), jnp.float32),
        ],
        scratch_types=[
            pltpu.VMEM((_E, tw), jnp.float32),
            pltpu.VMEM((_G, tw), jnp.float32),
            pltpu.VMEM((_G, tw), jnp.int32),
            pltpu.VMEM((_E,), jnp.float32),
            pltpu.VMEM((_TOPK, tw), jnp.int32),
            pltpu.VMEM((_TOPK, tw), jnp.float32),
        ],
    )
    def route(scores_hbm, hv_hbm, hi_hbm, bias_hbm, oi_hbm, ow_hbm,
              sbuf, hvbuf, hibuf, bvmem, oi, ow):
        wid = lax.axis_index("s") * nc + lax.axis_index("c")
        base = wid * tw
        pltpu.sync_copy(scores_hbm.at[:, pl.ds(base, tw)], sbuf)
        pltpu.sync_copy(hv_hbm.at[:, pl.ds(base, tw)], hvbuf)
        pltpu.sync_copy(hi_hbm.at[:, pl.ds(base, tw)], hibuf)
        pltpu.sync_copy(bias_hbm, bvmem)
        lanes = lax.iota(jnp.int32, _L)
        neg = jnp.full((_L,), -1.0, jnp.float32)

        def slab_body(i, carry):
            off = pl.multiple_of(i * _L, _L)
            col = off + lanes
            hv = [hvbuf[g, pl.ds(off, _L)] for g in range(_G)]
            hi = [hibuf[g, pl.ds(off, _L)] for g in range(_G)]
            den = jnp.zeros((_L,), jnp.float32)
            sel_i = [None] * _TOPK
            sel_w = [None] * _TOPK
            for r in range(_TOPK):
                bv = hv[0]
                bi = hi[0]
                for g in range(1, _G):
                    take = hv[g] > bv
                    bi = jnp.where(take, hi[g], bi)
                    bv = jnp.maximum(bv, hv[g])
                w_r = bv - plsc.load_gather(bvmem, [bi])
                den = den + w_r
                sel_i[r] = bi
                sel_w[r] = w_r
                if r == _TOPK - 1:
                    break
                plsc.store_scatter(sbuf, [bi, col], neg)
                gbase = jnp.bitwise_and(bi, jnp.int32(-_GS))
                nv = neg
                ni = gbase
                for j in range(_GS):
                    e = gbase + j
                    c = plsc.load_gather(sbuf, [e, col])
                    take = c > nv
                    nv = jnp.maximum(nv, c)
                    ni = jnp.where(take, e, ni)
                wg = lax.shift_right_logical(bi, 3)
                for g in range(_G):
                    hit = wg == g
                    hv[g] = jnp.where(hit, nv, hv[g])
                    hi[g] = jnp.where(hit, ni, hi[g])
            f = jnp.float32(_SCALE) / (den + jnp.float32(1e-20))
            for r in range(_TOPK):
                oi[r, pl.ds(off, _L)] = sel_i[r]
                ow[r, pl.ds(off, _L)] = sel_w[r] * f
            return carry

        lax.fori_loop(0, nslab, slab_body, 0)
        pltpu.sync_copy(oi, oi_hbm.at[:, pl.ds(base, tw)])
        pltpu.sync_copy(ow, ow_hbm.at[:, pl.ds(base, tw)])

    return route(scores_t, hv0, hi0, bias)


def kernel(hidden_states, weight, e_score_correction_bias):
    bsz, seq_len, h = hidden_states.shape
    t = bsz * seq_len
    x = hidden_states.reshape(t, h).astype(jnp.float32)
    w = weight.astype(jnp.float32)
    b = e_score_correction_bias.astype(jnp.float32)
    scores_t, hv0, hi0 = _gate_scores_t(x, w, b)
    oi, ow = _route_sc(scores_t, hv0, hi0, b)
    return oi.T, ow.T
